# baseline (speedup 1.0000x reference)
import functools

import jax
import jax.numpy as jnp
from jax import lax
from jax.experimental import pallas as pl
from jax.experimental.pallas import tpu as pltpu
from jax.experimental.pallas import tpu_sc as plsc

N_TOK = 1024
D_FEAT = 256
K_TOP = 64
N_FRAME = 32


FRAMES_PER_STEP = 8


def _one_frame(xb, wkq, base):
    proj = jnp.dot(xb, wkq, preferred_element_type=jnp.float32)
    kproj = proj[:, 0:4]
    qproj = proj[:, 4:8]

    h = jax.lax.dot_general(
        kproj, qproj,
        dimension_numbers=(((1,), (1,)), ((), ())),
        preferred_element_type=jnp.float32,
    )

    scale = 1.0 / jnp.sqrt(jnp.float32(D_FEAT))
    m = jnp.max(h, axis=1, keepdims=True)
    e = jnp.exp(scale * (h - m))
    s = jnp.sum(e, axis=1, keepdims=True)
    a = jnp.sum(e / s, axis=0, keepdims=True)

    a_col = jnp.transpose(a)
    i_col = jax.lax.broadcasted_iota(jnp.int32, (N_TOK, 1), 0)
    i_row = jax.lax.broadcasted_iota(jnp.int32, (1, N_TOK), 1)
    cmp = ((a_col > a) | ((a_col == a) & (i_col < i_row))).astype(jnp.float32)
    ones_row = jnp.ones((1, N_TOK), jnp.float32)
    rank = jnp.dot(ones_row, cmp, preferred_element_type=jnp.float32)
    rank_i = rank.astype(jnp.int32)
    r_iota = jax.lax.broadcasted_iota(jnp.int32, (K_TOP, N_TOK), 0)
    p = (r_iota == rank_i).astype(jnp.int32)

    token = jax.lax.broadcasted_iota(jnp.int32, (K_TOP, N_TOK), 1)
    return jnp.sum(p * (token + base), axis=1, keepdims=True)


def _topk_indices(xf, wkq, frame_offset, n_frames):
    def body(x_ref, wkq_ref, idx_ref):
        step = pl.program_id(0)
        wkq_v = wkq_ref[...]
        for f in range(FRAMES_PER_STEP):
            base = (frame_offset + step * FRAMES_PER_STEP + f) * N_TOK
            idx_ref[f] = _one_frame(x_ref[f], wkq_v, base)

    return pl.pallas_call(
        body,
        grid=(n_frames // FRAMES_PER_STEP,),
        in_specs=[
            pl.BlockSpec((FRAMES_PER_STEP, N_TOK, D_FEAT), lambda i: (i, 0, 0)),
            pl.BlockSpec((D_FEAT, 8), lambda i: (0, 0)),
        ],
        out_specs=pl.BlockSpec((FRAMES_PER_STEP, K_TOP, 1), lambda i: (i, 0, 0)),
        out_shape=jax.ShapeDtypeStruct((n_frames, K_TOP, 1), jnp.int32),
        compiler_params=pltpu.CompilerParams(
            dimension_semantics=("arbitrary",),
        ),
    )(xf, wkq)


def _make_sc_gather(b_total):
    info = plsc.get_sparse_core_info()
    nc, ns = info.num_cores, info.num_subcores
    nw = nc * ns
    b_per_w = b_total // nw
    mesh = plsc.VectorSubcoreMesh(core_axis_name="c", subcore_axis_name="s")

    @functools.partial(
        pl.kernel,
        mesh=mesh,
        out_type=jax.ShapeDtypeStruct((b_total, D_FEAT), jnp.float32),
        scratch_types=[
            pltpu.VMEM((b_per_w,), jnp.int32),
            pltpu.VMEM((b_per_w, D_FEAT), jnp.float32),
            pltpu.SemaphoreType.DMA,
        ],
    )
    def sc_gather(rows_hbm, idx_hbm, out_hbm, idx_v, rows_v, sem):
        wid = lax.axis_index("s") * nc + lax.axis_index("c")
        base = wid * b_per_w
        pltpu.sync_copy(idx_hbm.at[pl.ds(base, b_per_w)], idx_v)
        pltpu.async_copy(rows_hbm.at[idx_v], rows_v, sem).wait()
        pltpu.sync_copy(rows_v, out_hbm.at[pl.ds(base, b_per_w)])

    return sc_gather


N_CHUNK = 2


def kernel(x, wk, wq):
    N, T, n, d_in = x.shape
    xf = x.reshape(N * T, n, d_in)
    wkq = jnp.concatenate([wk, wq], axis=1)
    rows = xf.reshape(N_FRAME * N_TOK, d_in)
    fpc = N_FRAME // N_CHUNK
    gather = _make_sc_gather(fpc * K_TOP)
    outs = []
    for c in range(N_CHUNK):
        xc = jax.lax.slice_in_dim(xf, c * fpc, (c + 1) * fpc, axis=0)
        idx = _topk_indices(xc, wkq, c * fpc, fpc).reshape(fpc * K_TOP)
        outs.append(gather(rows, idx))
    out = jnp.concatenate(outs, axis=0)
    return out.reshape(N, T, K_TOP, d_in)


# speedup vs baseline: 1.4371x; 1.4371x over previous
import functools

import jax
import jax.numpy as jnp
from jax import lax
from jax.experimental import pallas as pl
from jax.experimental.pallas import tpu as pltpu
from jax.experimental.pallas import tpu_sc as plsc

N_TOK = 1024
D_FEAT = 256
K_TOP = 64
N_FRAME = 32


FRAMES_PER_STEP = 8


def _one_frame(xb, wkq, base):
    proj = jnp.dot(xb, wkq, preferred_element_type=jnp.float32)
    kproj = proj[:, 0:4]
    qproj = proj[:, 4:8]

    h = jax.lax.dot_general(
        kproj, qproj,
        dimension_numbers=(((1,), (1,)), ((), ())),
        preferred_element_type=jnp.float32,
    )

    scale = 1.0 / jnp.sqrt(jnp.float32(D_FEAT))
    m = jnp.max(h, axis=1, keepdims=True)
    e = jnp.exp(scale * (h - m))
    s = jnp.sum(e, axis=1, keepdims=True)
    a = jnp.sum(e / s, axis=0, keepdims=True)

    a_col = jnp.transpose(a)
    i_col = jax.lax.broadcasted_iota(jnp.int32, (N_TOK, 1), 0)
    i_row = jax.lax.broadcasted_iota(jnp.int32, (1, N_TOK), 1)
    cmp = ((a_col > a) | ((a_col == a) & (i_col < i_row))).astype(jnp.float32)
    ones_row = jnp.ones((1, N_TOK), jnp.float32)
    rank = jnp.dot(ones_row, cmp, preferred_element_type=jnp.float32)
    rank_i = rank.astype(jnp.int32)
    r_iota = jax.lax.broadcasted_iota(jnp.int32, (K_TOP, N_TOK), 0)
    p = (r_iota == rank_i).astype(jnp.int32)

    token = jax.lax.broadcasted_iota(jnp.int32, (K_TOP, N_TOK), 1)
    return jnp.sum(p * (token + base), axis=1, keepdims=True)


def _topk_indices(xf, wkq, frame_offset, n_frames):
    def body(x_ref, wkq_ref, idx_ref):
        step = pl.program_id(0)
        wkq_v = wkq_ref[...]
        for f in range(FRAMES_PER_STEP):
            base = (frame_offset + step * FRAMES_PER_STEP + f) * N_TOK
            idx_ref[f] = _one_frame(x_ref[f], wkq_v, base)

    return pl.pallas_call(
        body,
        grid=(n_frames // FRAMES_PER_STEP,),
        in_specs=[
            pl.BlockSpec((FRAMES_PER_STEP, N_TOK, D_FEAT), lambda i: (i, 0, 0)),
            pl.BlockSpec((D_FEAT, 8), lambda i: (0, 0)),
        ],
        out_specs=pl.BlockSpec((FRAMES_PER_STEP, K_TOP, 1), lambda i: (i, 0, 0)),
        out_shape=jax.ShapeDtypeStruct((n_frames, K_TOP, 1), jnp.int32),
        compiler_params=pltpu.CompilerParams(
            dimension_semantics=("arbitrary",),
        ),
    )(xf, wkq)


def _make_sc_gather(b_total):
    info = plsc.get_sparse_core_info()
    nc, ns = info.num_cores, info.num_subcores
    nw = nc * ns
    b_per_w = b_total // nw
    mesh = plsc.VectorSubcoreMesh(core_axis_name="c", subcore_axis_name="s")

    @functools.partial(
        pl.kernel,
        mesh=mesh,
        out_type=jax.ShapeDtypeStruct((b_total, D_FEAT), jnp.float32),
        scratch_types=[
            pltpu.VMEM((b_per_w,), jnp.int32),
            pltpu.VMEM((b_per_w, D_FEAT), jnp.float32),
            pltpu.SemaphoreType.DMA,
        ],
    )
    def sc_gather(rows_hbm, idx_hbm, out_hbm, idx_v, rows_v, sem):
        wid = lax.axis_index("s") * nc + lax.axis_index("c")
        base = wid * b_per_w
        pltpu.sync_copy(idx_hbm.at[pl.ds(base, b_per_w)], idx_v)
        pltpu.async_copy(rows_hbm.at[idx_v], rows_v, sem).wait()
        pltpu.sync_copy(rows_v, out_hbm.at[pl.ds(base, b_per_w)])

    return sc_gather


def kernel(x, wk, wq):
    N, T, n, d_in = x.shape
    xf = x.reshape(N * T, n, d_in)
    wkq = jnp.concatenate([wk, wq], axis=1)
    rows = xf.reshape(N_FRAME * N_TOK, d_in)
    idx = _topk_indices(xf, wkq, 0, N_FRAME).reshape(N_FRAME * K_TOP)
    out = _make_sc_gather(N_FRAME * K_TOP)(rows, idx)
    return out.reshape(N, T, K_TOP, d_in)
